# baseline (speedup 1.0000x reference)
import jax
import jax.numpy as jnp
from jax.experimental import pallas as pl
from jax.experimental.pallas import tpu as pltpu

_K = 32


_L = 128


def _tree_min(a, L):
    s = a.shape[1]
    while s > L:
        s //= 2
        a = jnp.minimum(a[:, :s], a[:, s:2 * s])
    return a


def _tile_up(t, N):
    while t.shape[1] < N:
        t = jnp.concatenate([t, t], axis=1)
    return t


def _topk_body(samp_ref, xyzt_ref, idx_ref, d_scr):
    R = samp_ref.shape[1]
    N = xyzt_ref.shape[2]
    q = samp_ref[0]
    p = xyzt_ref[0]
    d_scr[:, :] = ((q[:, 0:1] - p[0:1, :]) ** 2
                   + (q[:, 1:2] - p[1:2, :]) ** 2
                   + (q[:, 2:3] - p[2:3, :]) ** 2)
    colidx = jax.lax.broadcasted_iota(jnp.int32, (R, N), 1)
    inf = jnp.float32(jnp.inf)
    bigc = jnp.int32(N)

    def _col_tree(a):
        s = a.shape[0]
        while s > 1:
            s //= 2
            a = jnp.minimum(a[:s], a[s:2 * s])
        return a

    def round_body(carry):
        rvT, rcT, _ = carry
        d = d_scr[:, :]
        colmin = _tree_min(d, _L)
        eq = d == _tile_up(colmin, N)
        cand = jnp.where(eq, colidx, bigc)
        fcol = _tree_min(cand, _L)
        newd = jnp.where(colidx == _tile_up(fcol, N), inf, d)
        d_scr[:, :] = newd
        pad = 2 * _L - _K - _L
        cvT = jnp.concatenate(
            [rvT, colmin.T, jnp.full((pad, R), inf, jnp.float32)], axis=0)
        ccT = jnp.concatenate(
            [rcT, fcol.T, jnp.full((pad, R), bigc, jnp.int32)], axis=0)
        nrv, nrc = [], []
        for _ in range(_K):
            mv = _col_tree(cvT)
            mc = _col_tree(jnp.where(cvT == mv, ccT, bigc))
            nrv.append(mv)
            nrc.append(mc)
            cvT = jnp.where((cvT == mv) & (ccT == mc), inf, cvT)
        rvT2 = jnp.concatenate(nrv, axis=0)
        rcT2 = jnp.concatenate(nrc, axis=0)
        minremT = _col_tree(_tree_min(newd, _L).T)
        vT = rvT2
        s = _K
        while s > 1:
            s //= 2
            vT = jnp.maximum(vT[:s], vT[s:2 * s])
        go = jnp.any(minremT <= vT)
        return (rvT2, rcT2, go)

    init = (jnp.full((_K, R), inf, jnp.float32),
            jnp.full((_K, R), bigc, jnp.int32),
            jnp.bool_(True))
    _, rcT, _ = jax.lax.while_loop(lambda c: c[2], round_body, init)
    idx_ref[0] = rcT.T


def _gather_body(idx_ref, x_ref, out_ref):
    S = idx_ref.shape[1]

    def qstep(i, carry):
        acc = x_ref[0, idx_ref[0, i, 0], :]
        for k in range(1, _K):
            acc = jnp.maximum(acc, x_ref[0, idx_ref[0, i, k], :])
        out_ref[0, i, :] = acc
        return carry

    jax.lax.fori_loop(0, S, qstep, 0)


def kernel(input, batch_sample_xyz, sampling):
    B, N, D = input.shape
    M = sampling.shape[1]
    xyzt = jnp.transpose(batch_sample_xyz, (0, 2, 1))

    R = min(128, M)
    idx = pl.pallas_call(
        _topk_body,
        grid=(B, M // R),
        in_specs=[
            pl.BlockSpec((1, R, 3), lambda b, i: (b, i, 0)),
            pl.BlockSpec((1, 3, N), lambda b, i: (b, 0, 0)),
        ],
        out_specs=pl.BlockSpec((1, R, _K), lambda b, i: (b, i, 0)),
        out_shape=jax.ShapeDtypeStruct((B, M, _K), jnp.int32),
        scratch_shapes=[pltpu.VMEM((R, N), jnp.float32)],
    )(sampling, xyzt)

    S = min(256, M)
    out = pl.pallas_call(
        _gather_body,
        grid=(B, M // S),
        in_specs=[
            pl.BlockSpec((1, S, _K), lambda b, i: (b, i, 0),
                         memory_space=pltpu.SMEM),
            pl.BlockSpec((1, N, D), lambda b, i: (b, 0, 0)),
        ],
        out_specs=pl.BlockSpec((1, S, D), lambda b, i: (b, i, 0)),
        out_shape=jax.ShapeDtypeStruct((B, M, D), jnp.float32),
    )(idx, input)
    return out


# speedup vs baseline: 1.1188x; 1.1188x over previous
import functools

import jax
import jax.numpy as jnp
from jax import lax
from jax.experimental import pallas as pl
from jax.experimental.pallas import tpu as pltpu
from jax.experimental.pallas import tpu_sc as plsc

_K = 32


_L = 128


def _tree_min(a, L):
    s = a.shape[1]
    while s > L:
        s //= 2
        a = jnp.minimum(a[:, :s], a[:, s:2 * s])
    return a


def _tile_up(t, N):
    while t.shape[1] < N:
        t = jnp.concatenate([t, t], axis=1)
    return t


def _topk_body(samp_ref, xyzt_ref, idx_ref, d_scr):
    R = samp_ref.shape[1]
    N = xyzt_ref.shape[2]
    q = samp_ref[0]
    p = xyzt_ref[0]
    d_scr[:, :] = ((q[:, 0:1] - p[0:1, :]) ** 2
                   + (q[:, 1:2] - p[1:2, :]) ** 2
                   + (q[:, 2:3] - p[2:3, :]) ** 2)
    colidx = jax.lax.broadcasted_iota(jnp.int32, (R, N), 1)
    inf = jnp.float32(jnp.inf)
    bigc = jnp.int32(N)

    def _col_tree(a):
        s = a.shape[0]
        while s > 1:
            s //= 2
            a = jnp.minimum(a[:s], a[s:2 * s])
        return a

    def round_body(carry):
        rvT, rcT, _ = carry
        d = d_scr[:, :]
        colmin = _tree_min(d, _L)
        eq = d == _tile_up(colmin, N)
        cand = jnp.where(eq, colidx, bigc)
        fcol = _tree_min(cand, _L)
        newd = jnp.where(colidx == _tile_up(fcol, N), inf, d)
        d_scr[:, :] = newd
        pad = 2 * _L - _K - _L
        cvT = jnp.concatenate(
            [rvT, colmin.T, jnp.full((pad, R), inf, jnp.float32)], axis=0)
        ccT = jnp.concatenate(
            [rcT, fcol.T, jnp.full((pad, R), bigc, jnp.int32)], axis=0)
        nrv, nrc = [], []
        for _ in range(_K):
            mv = _col_tree(cvT)
            mc = _col_tree(jnp.where(cvT == mv, ccT, bigc))
            nrv.append(mv)
            nrc.append(mc)
            cvT = jnp.where((cvT == mv) & (ccT == mc), inf, cvT)
        rvT2 = jnp.concatenate(nrv, axis=0)
        rcT2 = jnp.concatenate(nrc, axis=0)
        minremT = _col_tree(_tree_min(newd, _L).T)
        vT = rvT2
        s = _K
        while s > 1:
            s //= 2
            vT = jnp.maximum(vT[:s], vT[s:2 * s])
        go = jnp.any(minremT <= vT)
        return (rvT2, rcT2, go)

    init = (jnp.full((_K, R), inf, jnp.float32),
            jnp.full((_K, R), bigc, jnp.int32),
            jnp.bool_(True))
    _, rcT, _ = jax.lax.while_loop(lambda c: c[2], round_body, init)
    idx_ref[0] = rcT.T


_NW = 32
_QC = 4


def _sc_pool(table, idxflat, BM, D):
    QW = BM // _NW
    mesh = plsc.VectorSubcoreMesh(core_axis_name="c", subcore_axis_name="s")

    @functools.partial(
        pl.kernel,
        out_type=jax.ShapeDtypeStruct((BM, D), jnp.float32),
        mesh=mesh,
        scratch_types=[
            pltpu.VMEM((QW * _K,), jnp.int32),
            pltpu.VMEM((_QC * _K, 128), jnp.float32),
            pltpu.VMEM((_QC * _K, 128), jnp.float32),
            pltpu.VMEM((2 * _QC, D), jnp.float32),
            pltpu.SemaphoreType.DMA,
            pltpu.SemaphoreType.DMA,
        ],
    )
    def k(table_hbm, idx_hbm, out_hbm, idx_v, ra, rb, oc, sa, sb):
        wid = lax.axis_index("s") * 2 + lax.axis_index("c")
        base = pl.multiple_of(wid * (QW * _K), 8)
        pltpu.sync_copy(idx_hbm.at[pl.ds(base, QW * _K)], idx_v)

        def jstep(j, carry):
            offa = pl.multiple_of(j * (2 * _QC * _K), 8)
            ca = pltpu.async_copy(
                table_hbm.at[idx_v.at[pl.ds(offa, _QC * _K)]], ra, sa)
            cb = pltpu.async_copy(
                table_hbm.at[idx_v.at[pl.ds(offa + _QC * _K, _QC * _K)]],
                rb, sb)
            ca.wait()
            for q in range(_QC):
                for g in range(D // 16):
                    acc = ra[q * _K, pl.ds(g * 16, 16)]
                    for r in range(1, _K):
                        acc = jnp.maximum(acc, ra[q * _K + r, pl.ds(g * 16, 16)])
                    oc[q, pl.ds(g * 16, 16)] = acc
            cb.wait()
            for q in range(_QC):
                for g in range(D // 16):
                    acc = rb[q * _K, pl.ds(g * 16, 16)]
                    for r in range(1, _K):
                        acc = jnp.maximum(acc, rb[q * _K + r, pl.ds(g * 16, 16)])
                    oc[_QC + q, pl.ds(g * 16, 16)] = acc
            obase = pl.multiple_of(wid * QW + j * (2 * _QC), 8)
            pltpu.sync_copy(oc, out_hbm.at[pl.ds(obase, 2 * _QC)])
            return carry

        lax.fori_loop(0, QW // (2 * _QC), jstep, 0)

    return k(table, idxflat)


def kernel(input, batch_sample_xyz, sampling):
    B, N, D = input.shape
    M = sampling.shape[1]
    xyzt = jnp.transpose(batch_sample_xyz, (0, 2, 1))

    R = min(128, M)
    idx = pl.pallas_call(
        _topk_body,
        grid=(B, M // R),
        in_specs=[
            pl.BlockSpec((1, R, 3), lambda b, i: (b, i, 0)),
            pl.BlockSpec((1, 3, N), lambda b, i: (b, 0, 0)),
        ],
        out_specs=pl.BlockSpec((1, R, _K), lambda b, i: (b, i, 0)),
        out_shape=jax.ShapeDtypeStruct((B, M, _K), jnp.int32),
        scratch_shapes=[pltpu.VMEM((R, N), jnp.float32)],
    )(sampling, xyzt)

    table = jnp.pad(input.reshape(B * N, D), ((0, 0), (0, 128 - D)))
    idxflat = (idx + (jnp.arange(B, dtype=jnp.int32) * N)[:, None, None]
               ).reshape(B * M * _K)
    out = _sc_pool(table, idxflat, B * M, D)
    return out.reshape(B, M, D)
